# baseline (speedup 1.0000x reference)
import functools

import jax
import jax.numpy as jnp
from jax import lax
from jax.experimental import pallas as pl
from jax.experimental.pallas import tpu as pltpu
from jax.experimental.pallas import tpu_sc as plsc

N = 10000
E = 320000
D_IN = 128
HID = 128
HEADS = 8

NP = 10240
BN = 1280
NB = NP // BN
NC = 2
NS = 16
L = 16
BLK = 128
NBLKS = E // BLK
ROWS_PER_TILE = NP // NS


def _tc0_body(x_ref, w0_ref, asrc_ref, adst_ref, h0_ref, asT_ref, adT_ref):
    h = pl.program_id(0)
    hb = jnp.dot(x_ref[...], w0_ref[...], preferred_element_type=jnp.float32)
    h0_ref[0] = hb
    sel = lax.broadcasted_iota(jnp.int32, (HEADS, 1), 0) == h
    arow_s = jnp.sum(jnp.where(sel, asrc_ref[...], 0.0), axis=0, keepdims=True)
    arow_d = jnp.sum(jnp.where(sel, adst_ref[...], 0.0), axis=0, keepdims=True)
    asT_ref[0, 0] = jnp.sum(hb * arow_s, axis=1)
    adT_ref[0, 0] = jnp.sum(hb * arow_d, axis=1)


def _tc0(xp, W0, att_src0, att_dst0):
    return pl.pallas_call(
        _tc0_body,
        grid=(HEADS, NB),
        in_specs=[
            pl.BlockSpec((BN, D_IN), lambda h, nb: (nb, 0)),
            pl.BlockSpec((D_IN, HID), lambda h, nb: (0, h)),
            pl.BlockSpec((HEADS, HID), lambda h, nb: (0, 0)),
            pl.BlockSpec((HEADS, HID), lambda h, nb: (0, 0)),
        ],
        out_specs=[
            pl.BlockSpec((1, BN, HID), lambda h, nb: (h, nb, 0)),
            pl.BlockSpec((1, 1, BN), lambda h, nb: (h, 0, nb)),
            pl.BlockSpec((1, 1, BN), lambda h, nb: (h, 0, nb)),
        ],
        out_shape=[
            jax.ShapeDtypeStruct((HEADS, NP, HID), jnp.float32),
            jax.ShapeDtypeStruct((HEADS, 1, NP), jnp.float32),
            jax.ShapeDtypeStruct((HEADS, 1, NP), jnp.float32),
        ],
    )(xp, W0, att_src0, att_dst0)


EPAD = 7680
EP = E + EPAD
NBLKP = EP // BLK
SBB = 5
SBE = SBB * BLK


def _zero_rows(rows, n):
    zv = jnp.zeros((L,), jnp.float32)

    def body(r, c):
        for j in range(HID // L):
            rows[r, pl.ds(j * L, L)] = zv
        return c

    lax.fori_loop(0, n, body, 0)


def _zero_tab(tab):
    zv = jnp.zeros((L,), jnp.float32)

    def body(i, c):
        tab[pl.ds(i * L, L)] = zv
        return c

    lax.fori_loop(0, NP // L, body, 0)


def _sc_body(layer0, src_hbm, dst_hbm, featb_hbm, asT_hbm, adT_hbm,
             num_hbm, den_hbm,
             accum, asrc_tab, adst_tab, den_tab, srcsb, dstsb,
             rows_f32, gidx, dstb, wb):
    cid = lax.axis_index("c")
    sid = lax.axis_index("s")
    rsl = pl.ds(sid * ROWS_PER_TILE, ROWS_PER_TILE)

    if layer0:
        passes = HEADS // NC
        bpt = NBLKP // NS
        base = sid * bpt
    else:
        passes = 1
        bpt = NBLKP // (NS * NC)
        base = (sid * NC + cid) * bpt

    for hp in range(passes):
        if layer0:
            h = (NC * hp + cid).astype(jnp.int32)
            row_off = h * NP
        else:
            h = jnp.int32(0)
            row_off = jnp.int32(0)
        _zero_rows(rows_f32, BLK)
        for q in range(ROWS_PER_TILE // BLK):
            pltpu.sync_copy(
                rows_f32,
                accum.at[pl.ds(sid * ROWS_PER_TILE + q * BLK, BLK)])
        _zero_tab(den_tab)
        pltpu.sync_copy(asT_hbm.at[h], asrc_tab)
        pltpu.sync_copy(adT_hbm.at[h], adst_tab)
        plsc.subcore_barrier()

        def blk_body(i, c):
            m = lax.rem(i, SBB)

            @pl.when(m == 0)
            def _():
                goff = (base + i) * BLK
                pltpu.sync_copy(src_hbm.at[pl.ds(goff, SBE)], srcsb)
                pltpu.sync_copy(dst_hbm.at[pl.ds(goff, SBE)], dstsb)

            moff = m * BLK
            for k in range(BLK // L):
                sb_sl = pl.ds(moff + k * L, L)
                sl = pl.ds(k * L, L)
                s16 = srcsb[sb_sl]
                d16 = dstsb[sb_sl]
                a1 = plsc.load_gather(asrc_tab, [s16])
                a2 = plsc.load_gather(adst_tab, [d16])
                al = a1 + a2
                al = jnp.where(al >= 0.0, al, 0.2 * al)
                w16 = jnp.exp(al)
                wb[sl] = w16
                gidx[sl] = s16 + row_off
                dstb[sl] = d16
                plsc.addupdate_scatter(den_tab, [d16], w16)

            pltpu.sync_copy(featb_hbm.at[gidx], rows_f32)

            def sbody(k, c2):
                w16 = wb[pl.ds(k * L, L)]
                for ll in range(L):
                    r = k * L + ll
                    wl = w16[ll]
                    for j in range(HID // L):
                        sl2 = pl.ds(j * L, L)
                        rows_f32[r, sl2] = rows_f32[r, sl2] * wl
                return c2

            lax.fori_loop(0, BLK // L, sbody, 0)
            pltpu.sync_copy(rows_f32, accum.at[dstb], add=True)
            return c

        lax.fori_loop(0, bpt, blk_body, 0)
        plsc.subcore_barrier()
        if layer0:
            pltpu.sync_copy(accum.at[rsl], num_hbm.at[h, rsl])
            pltpu.sync_copy(den_tab, den_hbm.at[h, sid])
        else:
            pltpu.sync_copy(accum.at[rsl], num_hbm.at[cid, rsl])
            pltpu.sync_copy(den_tab, den_hbm.at[cid, sid])
        plsc.subcore_barrier()


def _sc_edge(layer0, src, dst, featb, asT, adT):
    mesh = plsc.VectorSubcoreMesh(core_axis_name="c", subcore_axis_name="s",
                                  num_cores=NC, num_subcores=NS)
    dim0 = HEADS if layer0 else NC
    f = pl.kernel(
        functools.partial(_sc_body, layer0),
        out_type=[
            jax.ShapeDtypeStruct((dim0, NP, HID), jnp.float32),
            jax.ShapeDtypeStruct((dim0, NS, NP), jnp.float32),
        ],
        mesh=mesh,
        compiler_params=pltpu.CompilerParams(needs_layout_passes=False),
        scratch_types=[
            pltpu.VMEM_SHARED((NP, HID), jnp.float32),
            pltpu.VMEM((NP,), jnp.float32),
            pltpu.VMEM((NP,), jnp.float32),
            pltpu.VMEM((NP,), jnp.float32),
            pltpu.VMEM((SBE,), jnp.int32),
            pltpu.VMEM((SBE,), jnp.int32),
            pltpu.VMEM((BLK, HID), jnp.float32),
            pltpu.VMEM((BLK,), jnp.int32),
            pltpu.VMEM((BLK,), jnp.int32),
            pltpu.VMEM((BLK,), jnp.float32),
        ],
    )
    return f(src, dst, featb, asT, adT)


def _unperm(a):
    n = a.shape[0]
    return a.reshape(n, HID // (2 * L), 2, L).transpose(0, 1, 3, 2).reshape(
        n, HID)


def _tcmid_body(num_ref, den_ref, b0_ref, w1_ref, a1s_ref, a1d_ref,
                h1_ref, asT_ref, adT_ref):
    den = jnp.sum(den_ref[...], axis=1)
    acc = jnp.zeros((BN, HID), jnp.float32)
    for h in range(HEADS):
        v = num_ref[h] / (den[h][:, None] + 1e-16) + b0_ref[h][None, :]
        v = jnp.where(v > 0.0, v, jnp.exp(v) - 1.0)
        acc = acc + jnp.dot(v, w1_ref[h], preferred_element_type=jnp.float32)
    h1_ref[...] = acc
    asT_ref[0] = jnp.sum(acc * a1s_ref[...], axis=1)
    adT_ref[0] = jnp.sum(acc * a1d_ref[...], axis=1)


def _tcmid(num0, den0, b0r, W1r, att_src1, att_dst1):
    return pl.pallas_call(
        _tcmid_body,
        grid=(NB,),
        in_specs=[
            pl.BlockSpec((HEADS, BN, HID), lambda nb: (0, nb, 0)),
            pl.BlockSpec((HEADS, NS, BN), lambda nb: (0, 0, nb)),
            pl.BlockSpec((HEADS, HID), lambda nb: (0, 0)),
            pl.BlockSpec((HEADS, HID, HID), lambda nb: (0, 0, 0)),
            pl.BlockSpec((1, HID), lambda nb: (0, 0)),
            pl.BlockSpec((1, HID), lambda nb: (0, 0)),
        ],
        out_specs=[
            pl.BlockSpec((BN, HID), lambda nb: (nb, 0)),
            pl.BlockSpec((1, BN), lambda nb: (0, nb)),
            pl.BlockSpec((1, BN), lambda nb: (0, nb)),
        ],
        out_shape=[
            jax.ShapeDtypeStruct((NP, HID), jnp.float32),
            jax.ShapeDtypeStruct((1, NP), jnp.float32),
            jax.ShapeDtypeStruct((1, NP), jnp.float32),
        ],
    )(num0, den0, b0r, W1r, att_src1, att_dst1)


def _tcfin_body(num_ref, den_ref, b1_ref, out_ref):
    den = jnp.sum(den_ref[...], axis=(0, 1))
    out_ref[...] = ((num_ref[0] + num_ref[1]) / (den[:, None] + 1e-16)
                    + b1_ref[...])


def _tcfin(num1, den1, b1r):
    return pl.pallas_call(
        _tcfin_body,
        grid=(NB,),
        in_specs=[
            pl.BlockSpec((NC, BN, HID), lambda nb: (0, nb, 0)),
            pl.BlockSpec((NC, NS, BN), lambda nb: (0, 0, nb)),
            pl.BlockSpec((1, HID), lambda nb: (0, 0)),
        ],
        out_specs=pl.BlockSpec((BN, HID), lambda nb: (nb, 0)),
        out_shape=jax.ShapeDtypeStruct((NP, HID), jnp.float32),
    )(num1, den1, b1r)


@jax.jit
def kernel(x, edge_index, W0, att_src0, att_dst0, b0, W1, att_src1, att_dst1, b1):
    xp = jnp.pad(x, ((0, NP - N), (0, 0)))
    src = jnp.pad(edge_index[0], (0, EPAD), constant_values=N)
    dst = jnp.pad(edge_index[1], (0, EPAD), constant_values=N)

    h0T, asT0, adT0 = _tc0(xp, W0, att_src0, att_dst0)
    h0flat = h0T.reshape(HEADS * NP, HID)
    num0, den0 = _sc_edge(True, src, dst, h0flat,
                          asT0.reshape(HEADS, NP), adT0.reshape(HEADS, NP))

    b0r = b0.reshape(HEADS, HID)
    W1r = W1.reshape(HEADS, HID, HID)
    h1, asT1, adT1 = _tcmid(num0, den0, b0r, W1r, att_src1, att_dst1)

    num1, den1 = _sc_edge(False, src, dst, h1, asT1, adT1)
    outp = _tcfin(num1, den1, b1.reshape(1, HID))
    return outp[:N]


# speedup vs baseline: 1.7305x; 1.7305x over previous
import functools

import jax
import jax.numpy as jnp
from jax import lax
from jax.experimental import pallas as pl
from jax.experimental.pallas import tpu as pltpu
from jax.experimental.pallas import tpu_sc as plsc

N = 10000
E = 320000
D_IN = 128
HID = 128
HEADS = 8

NP = 10240
BN = 1280
NB = NP // BN
NC = 2
NS = 16
L = 16
BLK = 128
NBLKS = E // BLK
ROWS_PER_TILE = NP // NS


def _tc0_body(x_ref, w0_ref, asrc_ref, adst_ref, h0_ref, asT_ref, adT_ref):
    h = pl.program_id(0)
    hb = jnp.dot(x_ref[...], w0_ref[...], preferred_element_type=jnp.float32)
    h0_ref[0] = hb
    sel = lax.broadcasted_iota(jnp.int32, (HEADS, 1), 0) == h
    arow_s = jnp.sum(jnp.where(sel, asrc_ref[...], 0.0), axis=0, keepdims=True)
    arow_d = jnp.sum(jnp.where(sel, adst_ref[...], 0.0), axis=0, keepdims=True)
    asT_ref[0, 0] = jnp.sum(hb * arow_s, axis=1)
    adT_ref[0, 0] = jnp.sum(hb * arow_d, axis=1)


def _tc0(xp, W0, att_src0, att_dst0):
    return pl.pallas_call(
        _tc0_body,
        grid=(HEADS, NB),
        in_specs=[
            pl.BlockSpec((BN, D_IN), lambda h, nb: (nb, 0)),
            pl.BlockSpec((D_IN, HID), lambda h, nb: (0, h)),
            pl.BlockSpec((HEADS, HID), lambda h, nb: (0, 0)),
            pl.BlockSpec((HEADS, HID), lambda h, nb: (0, 0)),
        ],
        out_specs=[
            pl.BlockSpec((1, BN, HID), lambda h, nb: (h, nb, 0)),
            pl.BlockSpec((1, 1, BN), lambda h, nb: (h, 0, nb)),
            pl.BlockSpec((1, 1, BN), lambda h, nb: (h, 0, nb)),
        ],
        out_shape=[
            jax.ShapeDtypeStruct((HEADS, NP, HID), jnp.float32),
            jax.ShapeDtypeStruct((HEADS, 1, NP), jnp.float32),
            jax.ShapeDtypeStruct((HEADS, 1, NP), jnp.float32),
        ],
    )(xp, W0, att_src0, att_dst0)


EPAD = 7680
EP = E + EPAD
NBLKP = EP // BLK
SBB = 5
SBE = SBB * BLK


def _zero_rows(rows, n):
    zv = jnp.zeros((L,), jnp.float32)

    def body(r, c):
        for j in range(HID // L):
            rows[r, pl.ds(j * L, L)] = zv
        return c

    lax.fori_loop(0, n, body, 0)


def _zero_tab(tab):
    zv = jnp.zeros((L,), jnp.float32)

    def body(i, c):
        tab[pl.ds(i * L, L)] = zv
        return c

    lax.fori_loop(0, NP // L, body, 0)


def _sc_body(layer0, src_hbm, dst_hbm, featb_hbm, asT_hbm, adT_hbm,
             num_hbm, den_hbm,
             accum, asrc_tab, adst_tab, den_tab, srcsb, dstsb,
             rows_f32, gidx, dstb, wb):
    cid = lax.axis_index("c")
    sid = lax.axis_index("s")
    rsl = pl.ds(sid * ROWS_PER_TILE, ROWS_PER_TILE)

    if layer0:
        passes = HEADS // NC
        bpt = NBLKP // NS
        base = sid * bpt
    else:
        passes = 1
        bpt = NBLKP // (NS * NC)
        base = (sid * NC + cid) * bpt

    for hp in range(passes):
        if layer0:
            h = (NC * hp + cid).astype(jnp.int32)
            row_off = h * NP
        else:
            h = jnp.int32(0)
            row_off = jnp.int32(0)
        _zero_rows(rows_f32, BLK)
        for q in range(ROWS_PER_TILE // BLK):
            pltpu.sync_copy(
                rows_f32,
                accum.at[pl.ds(sid * ROWS_PER_TILE + q * BLK, BLK)])
        _zero_tab(den_tab)
        pltpu.sync_copy(asT_hbm.at[h], asrc_tab)
        pltpu.sync_copy(adT_hbm.at[h], adst_tab)
        plsc.subcore_barrier()

        def blk_body(i, c):
            m = lax.rem(i, SBB)

            @pl.when(m == 0)
            def _():
                goff = (base + i) * BLK
                pltpu.sync_copy(src_hbm.at[pl.ds(goff, SBE)], srcsb)
                pltpu.sync_copy(dst_hbm.at[pl.ds(goff, SBE)], dstsb)

            moff = m * BLK
            for k in range(BLK // L):
                sb_sl = pl.ds(moff + k * L, L)
                sl = pl.ds(k * L, L)
                s16 = srcsb[sb_sl]
                d16 = dstsb[sb_sl]
                a1 = plsc.load_gather(asrc_tab, [s16])
                a2 = plsc.load_gather(adst_tab, [d16])
                al = a1 + a2
                al = jnp.where(al >= 0.0, al, 0.2 * al)
                w16 = jnp.exp(al)
                wb[sl] = w16
                gidx[sl] = s16 + row_off
                dstb[sl] = d16
                plsc.addupdate_scatter(den_tab, [d16], w16)

            pltpu.sync_copy(featb_hbm.at[gidx], rows_f32)

            def sbody(k, c2):
                w16 = wb[pl.ds(k * L, L)]
                for ll in range(L):
                    r = k * L + ll
                    wl = w16[ll]
                    for j in range(HID // L):
                        sl2 = pl.ds(j * L, L)
                        rows_f32[r, sl2] = rows_f32[r, sl2] * wl
                return c2

            lax.fori_loop(0, BLK // L, sbody, 0)
            pltpu.sync_copy(rows_f32, accum.at[dstb], add=True)
            return c

        lax.fori_loop(0, bpt, blk_body, 0)
        plsc.subcore_barrier()
        if layer0:
            pltpu.sync_copy(accum.at[rsl], num_hbm.at[h, rsl])
            pltpu.sync_copy(den_tab, den_hbm.at[h, sid])
        else:
            pltpu.sync_copy(accum.at[rsl], num_hbm.at[cid, rsl])
            pltpu.sync_copy(den_tab, den_hbm.at[cid, sid])
        plsc.subcore_barrier()


def _sc_edge(layer0, src, dst, featb, asT, adT):
    mesh = plsc.VectorSubcoreMesh(core_axis_name="c", subcore_axis_name="s",
                                  num_cores=NC, num_subcores=NS)
    dim0 = HEADS if layer0 else NC
    f = pl.kernel(
        functools.partial(_sc_body, layer0),
        out_type=[
            jax.ShapeDtypeStruct((dim0, NP, HID), jnp.float32),
            jax.ShapeDtypeStruct((dim0, NS, NP), jnp.float32),
        ],
        mesh=mesh,
        compiler_params=pltpu.CompilerParams(needs_layout_passes=False),
        scratch_types=[
            pltpu.VMEM_SHARED((NP, HID), jnp.float32),
            pltpu.VMEM((NP,), jnp.float32),
            pltpu.VMEM((NP,), jnp.float32),
            pltpu.VMEM((NP,), jnp.float32),
            pltpu.VMEM((SBE,), jnp.int32),
            pltpu.VMEM((SBE,), jnp.int32),
            pltpu.VMEM((BLK, HID), jnp.float32),
            pltpu.VMEM((BLK,), jnp.int32),
            pltpu.VMEM((BLK,), jnp.int32),
            pltpu.VMEM((BLK,), jnp.float32),
        ],
    )
    return f(src, dst, featb, asT, adT)


def _unperm(a):
    n = a.shape[0]
    return a.reshape(n, HID // (2 * L), 2, L).transpose(0, 1, 3, 2).reshape(
        n, HID)


def _tcmid_body(num_ref, den_ref, b0_ref, w1_ref, a1s_ref, a1d_ref,
                h1_ref, asT_ref, adT_ref):
    den = jnp.sum(den_ref[...], axis=1)
    acc = jnp.zeros((BN, HID), jnp.float32)
    for h in range(HEADS):
        v = num_ref[h] / (den[h][:, None] + 1e-16) + b0_ref[h][None, :]
        v = jnp.where(v > 0.0, v, jnp.exp(v) - 1.0)
        acc = acc + jnp.dot(v, w1_ref[h], preferred_element_type=jnp.float32)
    h1_ref[...] = acc
    asT_ref[0] = jnp.sum(acc * a1s_ref[...], axis=1)
    adT_ref[0] = jnp.sum(acc * a1d_ref[...], axis=1)


def _tcmid(num0, den0, b0r, W1r, att_src1, att_dst1):
    return pl.pallas_call(
        _tcmid_body,
        grid=(NB,),
        in_specs=[
            pl.BlockSpec((HEADS, BN, HID), lambda nb: (0, nb, 0)),
            pl.BlockSpec((HEADS, NS, BN), lambda nb: (0, 0, nb)),
            pl.BlockSpec((HEADS, HID), lambda nb: (0, 0)),
            pl.BlockSpec((HEADS, HID, HID), lambda nb: (0, 0, 0)),
            pl.BlockSpec((1, HID), lambda nb: (0, 0)),
            pl.BlockSpec((1, HID), lambda nb: (0, 0)),
        ],
        out_specs=[
            pl.BlockSpec((BN, HID), lambda nb: (nb, 0)),
            pl.BlockSpec((1, BN), lambda nb: (0, nb)),
            pl.BlockSpec((1, BN), lambda nb: (0, nb)),
        ],
        out_shape=[
            jax.ShapeDtypeStruct((NP, HID), jnp.float32),
            jax.ShapeDtypeStruct((1, NP), jnp.float32),
            jax.ShapeDtypeStruct((1, NP), jnp.float32),
        ],
    )(num0, den0, b0r, W1r, att_src1, att_dst1)


def _tcfin_body(num_ref, den_ref, b1_ref, out_ref):
    den = jnp.sum(den_ref[...], axis=(0, 1))
    out_ref[...] = ((num_ref[0] + num_ref[1]) / (den[:, None] + 1e-16)
                    + b1_ref[...])


def _tcfin(num1, den1, b1r):
    return pl.pallas_call(
        _tcfin_body,
        grid=(NB,),
        in_specs=[
            pl.BlockSpec((NC, BN, HID), lambda nb: (0, nb, 0)),
            pl.BlockSpec((NC, NS, BN), lambda nb: (0, 0, nb)),
            pl.BlockSpec((1, HID), lambda nb: (0, 0)),
        ],
        out_specs=pl.BlockSpec((BN, HID), lambda nb: (nb, 0)),
        out_shape=jax.ShapeDtypeStruct((NP, HID), jnp.float32),
    )(num1, den1, b1r)


@jax.jit
def kernel(x, edge_index, W0, att_src0, att_dst0, b0, W1, att_src1, att_dst1, b1):
    xp = jnp.pad(x, ((0, NP - N), (0, 0)))
    padn = N + jnp.arange(EPAD, dtype=jnp.int32) % (NP - N)
    src = jnp.concatenate([edge_index[0], padn])
    dst = jnp.concatenate([edge_index[1], padn])

    h0T, asT0, adT0 = _tc0(xp, W0, att_src0, att_dst0)
    h0flat = h0T.reshape(HEADS * NP, HID)
    num0, den0 = _sc_edge(True, src, dst, h0flat,
                          asT0.reshape(HEADS, NP), adT0.reshape(HEADS, NP))

    b0r = b0.reshape(HEADS, HID)
    W1r = W1.reshape(HEADS, HID, HID)
    h1, asT1, adT1 = _tcmid(num0, den0, b0r, W1r, att_src1, att_dst1)

    num1, den1 = _sc_edge(False, src, dst, h1, asT1, adT1)
    outp = _tcfin(num1, den1, b1.reshape(1, HID))
    return outp[:N]


# speedup vs baseline: 2.2817x; 1.3186x over previous
import functools

import jax
import jax.numpy as jnp
from jax import lax
from jax.experimental import pallas as pl
from jax.experimental.pallas import tpu as pltpu
from jax.experimental.pallas import tpu_sc as plsc

N = 10000
E = 320000
D_IN = 128
HID = 128
HEADS = 8

NP = 10240
BN = 1280
NB = NP // BN
NC = 2
NS = 16
L = 16
BLK = 128
NBLKS = E // BLK
ROWS_PER_TILE = NP // NS


def _tc0_body(x_ref, w0_ref, asrc_ref, adst_ref, h0_ref, asT_ref, adT_ref):
    h = pl.program_id(0)
    hb = jnp.dot(x_ref[...], w0_ref[...], preferred_element_type=jnp.float32)
    h0_ref[0] = hb
    sel = lax.broadcasted_iota(jnp.int32, (HEADS, 1), 0) == h
    arow_s = jnp.sum(jnp.where(sel, asrc_ref[...], 0.0), axis=0, keepdims=True)
    arow_d = jnp.sum(jnp.where(sel, adst_ref[...], 0.0), axis=0, keepdims=True)
    asT_ref[0, 0] = jnp.sum(hb * arow_s, axis=1)
    adT_ref[0, 0] = jnp.sum(hb * arow_d, axis=1)


def _tc0(xp, W0, att_src0, att_dst0):
    return pl.pallas_call(
        _tc0_body,
        grid=(HEADS, NB),
        in_specs=[
            pl.BlockSpec((BN, D_IN), lambda h, nb: (nb, 0)),
            pl.BlockSpec((D_IN, HID), lambda h, nb: (0, h)),
            pl.BlockSpec((HEADS, HID), lambda h, nb: (0, 0)),
            pl.BlockSpec((HEADS, HID), lambda h, nb: (0, 0)),
        ],
        out_specs=[
            pl.BlockSpec((1, BN, HID), lambda h, nb: (h, nb, 0)),
            pl.BlockSpec((1, 1, BN), lambda h, nb: (h, 0, nb)),
            pl.BlockSpec((1, 1, BN), lambda h, nb: (h, 0, nb)),
        ],
        out_shape=[
            jax.ShapeDtypeStruct((HEADS, NP, HID), jnp.float32),
            jax.ShapeDtypeStruct((HEADS, 1, NP), jnp.float32),
            jax.ShapeDtypeStruct((HEADS, 1, NP), jnp.float32),
        ],
    )(xp, W0, att_src0, att_dst0)


EPAD = 7680
EP = E + EPAD
EBLK = 64
NBLKP = EP // EBLK
SBB = 4
SBE = SBB * EBLK


def _zero_rows(rows, n):
    zv = jnp.zeros((L,), jnp.float32)

    def body(r, c):
        for j in range(HID // L):
            rows[r, pl.ds(j * L, L)] = zv
        return c

    lax.fori_loop(0, n, body, 0)


def _zero_tab(tab):
    zv = jnp.zeros((L,), jnp.float32)

    def body(i, c):
        tab[pl.ds(i * L, L)] = zv
        return c

    lax.fori_loop(0, NP // L, body, 0)


def _sc_body(layer0, src_hbm, dst_hbm, featb_hbm, asT_hbm, adT_hbm,
             num_hbm, den_hbm,
             accum, asrc_tab, adst_tab, den_tab, srcsb, dstsb,
             rows0, rows1, gidx0, gidx1, dstb0, dstb1, dstbS0, dstbS1,
             wb0, wb1, gsem, ssem, bsem):
    rows = (rows0, rows1)
    gidx = (gidx0, gidx1)
    dstb = (dstb0, dstb1)
    dstbS = (dstbS0, dstbS1)
    wb = (wb0, wb1)
    cid = lax.axis_index("c")
    sid = lax.axis_index("s")
    rsl = pl.ds(sid * ROWS_PER_TILE, ROWS_PER_TILE)

    if layer0:
        passes = HEADS // NC
        bpt = NBLKP // NS
        base = sid * bpt
    else:
        passes = 1
        bpt = NBLKP // (NS * NC)
        base = (sid * NC + cid) * bpt

    for hp in range(passes):
        if layer0:
            h = (NC * hp + cid).astype(jnp.int32)
            row_off = h * NP
        else:
            h = jnp.int32(0)
            row_off = jnp.int32(0)
        _zero_rows(rows0, EBLK)
        for q in range(ROWS_PER_TILE // EBLK):
            pltpu.sync_copy(
                rows0,
                accum.at[pl.ds(sid * ROWS_PER_TILE + q * EBLK, EBLK)])
        _zero_tab(den_tab)
        pltpu.sync_copy(asT_hbm.at[h], asrc_tab)
        pltpu.sync_copy(adT_hbm.at[h], adst_tab)
        plsc.subcore_barrier()

        def prep(i, p):
            m = lax.rem(i, SBB)
            sb = lax.rem(i // SBB, 2)
            moff = (sb * SBB + m) * EBLK
            for k in range(EBLK // L):
                sb_sl = pl.ds(moff + k * L, L)
                sl = pl.ds(k * L, L)
                s16 = srcsb[sb_sl]
                d16 = dstsb[sb_sl]
                a1 = plsc.load_gather(asrc_tab, [s16])
                a2 = plsc.load_gather(adst_tab, [d16])
                al = a1 + a2
                al = jnp.where(al >= 0.0, al, 0.2 * al)
                w16 = jnp.exp(al)
                wb[p][sl] = w16
                gidx[p][sl] = s16 + row_off
                dstb[p][sl] = d16
                plsc.addupdate_scatter(den_tab, [d16], w16)

        def scale(p):
            def sbody(k, c2):
                w16 = wb[p][pl.ds(k * L, L)]
                for ll in range(L):
                    r = k * L + ll
                    wl = w16[ll]
                    for j in range(HID // L):
                        sl2 = pl.ds(j * L, L)
                        rows[p][r, sl2] = rows[p][r, sl2] * wl
                return c2

            lax.fori_loop(0, EBLK // L, sbody, 0)

        def load_sb(blk0, sb):
            goff = (base + blk0) * EBLK
            pltpu.async_copy(src_hbm.at[pl.ds(goff, SBE)],
                             srcsb.at[pl.ds(sb * SBE, SBE)], bsem)
            pltpu.async_copy(dst_hbm.at[pl.ds(goff, SBE)],
                             dstsb.at[pl.ds(sb * SBE, SBE)], bsem)

        def enq_gather(i, p):
            pltpu.async_copy(featb_hbm.at[gidx[p]], rows[p], gsem)

        def wait_gather(p):
            pltpu.make_async_copy(featb_hbm.at[gidx[p]], rows[p], gsem).wait()

        def enq_scatter(p):
            for k in range(EBLK // L):
                sl = pl.ds(k * L, L)
                dstbS[p][sl] = dstb[p][sl]
            pltpu.async_copy(rows[p], accum.at[dstbS[p]], ssem, add=True)

        pltpu.sync_copy(src_hbm.at[pl.ds(base * EBLK, SBE)],
                        srcsb.at[pl.ds(0, SBE)])
        pltpu.sync_copy(dst_hbm.at[pl.ds(base * EBLK, SBE)],
                        dstsb.at[pl.ds(0, SBE)])
        load_sb(SBB, 1)
        prep(0, 0)
        enq_gather(0, 0)

        def sb_prefetch(i):
            @pl.when(lax.rem(i, SBB) == SBB - 2)
            def _():
                nxt = (i - lax.rem(i, SBB)) + 2 * SBB

                @pl.when(nxt < bpt)
                def _():
                    load_sb(nxt, lax.rem(nxt // SBB, 2))

        def pair_body(jj, c):
            i0 = 2 * jj
            wait_gather(0)
            prep(i0 + 1, 1)
            enq_gather(i0 + 1, 1)
            scale(0)
            enq_scatter(0)
            sb_prefetch(i0)
            wait_gather(1)

            @pl.when(i0 + 2 < bpt)
            def _():
                prep(i0 + 2, 0)
                enq_gather(i0 + 2, 0)
            scale(1)
            enq_scatter(1)
            sb_prefetch(i0 + 1)
            return c

        lax.fori_loop(0, bpt // 2, pair_body, 0)

        def drain_s(i, c):
            pltpu.make_async_copy(rows[0], accum.at[dstbS[0]], ssem).wait()
            return c

        lax.fori_loop(0, bpt, drain_s, 0)

        def drain_b(i, c):
            pltpu.make_async_copy(src_hbm.at[pl.ds(base * EBLK, SBE)],
                                  srcsb.at[pl.ds(0, SBE)], bsem).wait()
            return c

        nsb_async = 2 * ((bpt + SBB - 1) // SBB - 1)
        if nsb_async > 0:
            lax.fori_loop(0, nsb_async, drain_b, 0)
        plsc.subcore_barrier()
        if layer0:
            pltpu.sync_copy(accum.at[rsl], num_hbm.at[h, rsl])
            pltpu.sync_copy(den_tab, den_hbm.at[h, sid])
        else:
            pltpu.sync_copy(accum.at[rsl], num_hbm.at[cid, rsl])
            pltpu.sync_copy(den_tab, den_hbm.at[cid, sid])
        plsc.subcore_barrier()


def _sc_edge(layer0, src, dst, featb, asT, adT):
    mesh = plsc.VectorSubcoreMesh(core_axis_name="c", subcore_axis_name="s",
                                  num_cores=NC, num_subcores=NS)
    dim0 = HEADS if layer0 else NC
    f = pl.kernel(
        functools.partial(_sc_body, layer0),
        out_type=[
            jax.ShapeDtypeStruct((dim0, NP, HID), jnp.float32),
            jax.ShapeDtypeStruct((dim0, NS, NP), jnp.float32),
        ],
        mesh=mesh,
        compiler_params=pltpu.CompilerParams(needs_layout_passes=False),
        scratch_types=[
            pltpu.VMEM_SHARED((NP, HID), jnp.float32),
            pltpu.VMEM((NP,), jnp.float32),
            pltpu.VMEM((NP,), jnp.float32),
            pltpu.VMEM((NP,), jnp.float32),
            pltpu.VMEM((2 * SBE,), jnp.int32),
            pltpu.VMEM((2 * SBE,), jnp.int32),
            pltpu.VMEM((EBLK, HID), jnp.float32),
            pltpu.VMEM((EBLK, HID), jnp.float32),
            pltpu.VMEM((EBLK,), jnp.int32),
            pltpu.VMEM((EBLK,), jnp.int32),
            pltpu.VMEM((EBLK,), jnp.int32),
            pltpu.VMEM((EBLK,), jnp.int32),
            pltpu.VMEM((EBLK,), jnp.int32),
            pltpu.VMEM((EBLK,), jnp.int32),
            pltpu.VMEM((EBLK,), jnp.float32),
            pltpu.VMEM((EBLK,), jnp.float32),
            pltpu.SemaphoreType.DMA,
            pltpu.SemaphoreType.DMA,
            pltpu.SemaphoreType.DMA,
        ],
    )
    return f(src, dst, featb, asT, adT)


def _unperm(a):
    n = a.shape[0]
    return a.reshape(n, HID // (2 * L), 2, L).transpose(0, 1, 3, 2).reshape(
        n, HID)


def _tcmid_body(num_ref, den_ref, b0_ref, w1_ref, a1s_ref, a1d_ref,
                h1_ref, asT_ref, adT_ref):
    den = jnp.sum(den_ref[...], axis=1)
    acc = jnp.zeros((BN, HID), jnp.float32)
    for h in range(HEADS):
        v = num_ref[h] / (den[h][:, None] + 1e-16) + b0_ref[h][None, :]
        v = jnp.where(v > 0.0, v, jnp.exp(v) - 1.0)
        acc = acc + jnp.dot(v, w1_ref[h], preferred_element_type=jnp.float32)
    h1_ref[...] = acc
    asT_ref[0] = jnp.sum(acc * a1s_ref[...], axis=1)
    adT_ref[0] = jnp.sum(acc * a1d_ref[...], axis=1)


def _tcmid(num0, den0, b0r, W1r, att_src1, att_dst1):
    return pl.pallas_call(
        _tcmid_body,
        grid=(NB,),
        in_specs=[
            pl.BlockSpec((HEADS, BN, HID), lambda nb: (0, nb, 0)),
            pl.BlockSpec((HEADS, NS, BN), lambda nb: (0, 0, nb)),
            pl.BlockSpec((HEADS, HID), lambda nb: (0, 0)),
            pl.BlockSpec((HEADS, HID, HID), lambda nb: (0, 0, 0)),
            pl.BlockSpec((1, HID), lambda nb: (0, 0)),
            pl.BlockSpec((1, HID), lambda nb: (0, 0)),
        ],
        out_specs=[
            pl.BlockSpec((BN, HID), lambda nb: (nb, 0)),
            pl.BlockSpec((1, BN), lambda nb: (0, nb)),
            pl.BlockSpec((1, BN), lambda nb: (0, nb)),
        ],
        out_shape=[
            jax.ShapeDtypeStruct((NP, HID), jnp.float32),
            jax.ShapeDtypeStruct((1, NP), jnp.float32),
            jax.ShapeDtypeStruct((1, NP), jnp.float32),
        ],
    )(num0, den0, b0r, W1r, att_src1, att_dst1)


def _tcfin_body(num_ref, den_ref, b1_ref, out_ref):
    den = jnp.sum(den_ref[...], axis=(0, 1))
    out_ref[...] = ((num_ref[0] + num_ref[1]) / (den[:, None] + 1e-16)
                    + b1_ref[...])


def _tcfin(num1, den1, b1r):
    return pl.pallas_call(
        _tcfin_body,
        grid=(NB,),
        in_specs=[
            pl.BlockSpec((NC, BN, HID), lambda nb: (0, nb, 0)),
            pl.BlockSpec((NC, NS, BN), lambda nb: (0, 0, nb)),
            pl.BlockSpec((1, HID), lambda nb: (0, 0)),
        ],
        out_specs=pl.BlockSpec((BN, HID), lambda nb: (nb, 0)),
        out_shape=jax.ShapeDtypeStruct((NP, HID), jnp.float32),
    )(num1, den1, b1r)


@jax.jit
def kernel(x, edge_index, W0, att_src0, att_dst0, b0, W1, att_src1, att_dst1, b1):
    xp = jnp.pad(x, ((0, NP - N), (0, 0)))
    padn = N + jnp.arange(EPAD, dtype=jnp.int32) % (NP - N)
    src = jnp.concatenate([edge_index[0], padn])
    dst = jnp.concatenate([edge_index[1], padn])

    h0T, asT0, adT0 = _tc0(xp, W0, att_src0, att_dst0)
    h0flat = h0T.reshape(HEADS * NP, HID)
    num0, den0 = _sc_edge(True, src, dst, h0flat,
                          asT0.reshape(HEADS, NP), adT0.reshape(HEADS, NP))

    b0r = b0.reshape(HEADS, HID)
    W1r = W1.reshape(HEADS, HID, HID)
    h1, asT1, adT1 = _tcmid(num0, den0, b0r, W1r, att_src1, att_dst1)

    num1, den1 = _sc_edge(False, src, dst, h1, asT1, adT1)
    outp = _tcfin(num1, den1, b1.reshape(1, HID))
    return outp[:N]
